# Initial kernel scaffold; baseline (speedup 1.0000x reference)
import jax
import jax.numpy as jnp
from jax.experimental import pallas as pl


def kernel(prompt_mask, prompt):
    raise NotImplementedError("write your pallas kernel here")


# speedup vs baseline: 2.5536x; 2.5536x over previous
import functools

import jax
import jax.numpy as jnp
from jax import lax
from jax.experimental import pallas as pl
from jax.experimental.pallas import tpu as pltpu
from jax.experimental.pallas import tpu_sc as plsc

_POOL_SIZE = 1000
_LENGTH = 4
_EMBED_DIM = 128
_BATCH = 16384
_TOP_K = 5

_D = _LENGTH * _EMBED_DIM
_B_TOTAL = _BATCH * _TOP_K
_NC, _NS = 2, 16
_NW = _NC * _NS
_B_PER_W = _B_TOTAL // _NW
_CHUNK = 64
_N_CHUNKS = _B_PER_W // _CHUNK

_mesh = plsc.VectorSubcoreMesh(
    core_axis_name="c", subcore_axis_name="s",
    num_cores=_NC, num_subcores=_NS,
)


@functools.partial(
    pl.kernel,
    out_type=jax.ShapeDtypeStruct((_B_TOTAL, _D), jnp.float32),
    mesh=_mesh,
    scratch_types=[
        pltpu.VMEM((_N_CHUNKS, _CHUNK), jnp.int32),
        pltpu.VMEM((_CHUNK, _D), jnp.float32),
        pltpu.SemaphoreType.DMA,
    ],
)
def _gather_rows(idx_hbm, table_hbm, out_hbm, idx_v, buf, sem):
    wid = lax.axis_index("s") * _NC + lax.axis_index("c")
    pltpu.sync_copy(idx_hbm.at[wid], idx_v)
    base = wid * _B_PER_W

    @pl.loop(0, _N_CHUNKS)
    def _(g):
        pltpu.async_copy(table_hbm.at[idx_v.at[g]], buf, sem).wait()
        pltpu.sync_copy(buf, out_hbm.at[pl.ds(base + g * _CHUNK, _CHUNK)])


def kernel(prompt_mask, prompt):
    idx = prompt_mask.astype(jnp.int32).reshape(_NW, _N_CHUNKS, _CHUNK)
    table = prompt.reshape(_POOL_SIZE, _D)
    out = _gather_rows(idx, table)
    return out.reshape(_BATCH, _TOP_K * _LENGTH, _EMBED_DIM)


# speedup vs baseline: 2.6742x; 1.0472x over previous
import functools

import jax
import jax.numpy as jnp
from jax import lax
from jax.experimental import pallas as pl
from jax.experimental.pallas import tpu as pltpu
from jax.experimental.pallas import tpu_sc as plsc

_POOL_SIZE = 1000
_LENGTH = 4
_EMBED_DIM = 128
_BATCH = 16384
_TOP_K = 5

_D = _LENGTH * _EMBED_DIM
_B_TOTAL = _BATCH * _TOP_K
_NC, _NS = 2, 16
_NW = _NC * _NS
_B_PER_W = _B_TOTAL // _NW
_CHUNK = 64
_N_CHUNKS = _B_PER_W // _CHUNK

_mesh = plsc.VectorSubcoreMesh(
    core_axis_name="c", subcore_axis_name="s",
    num_cores=_NC, num_subcores=_NS,
)


_NBUF = 2


@functools.partial(
    pl.kernel,
    out_type=jax.ShapeDtypeStruct((_B_TOTAL, _D), jnp.float32),
    mesh=_mesh,
    scratch_types=[
        pltpu.VMEM((_N_CHUNKS, _CHUNK), jnp.int32),
        [pltpu.VMEM((_CHUNK, _D), jnp.float32) for _ in range(_NBUF)],
        [pltpu.SemaphoreType.DMA for _ in range(_NBUF)],
    ],
)
def _gather_rows(idx_hbm, table_hbm, out_hbm, idx_v, bufs, sems):
    wid = lax.axis_index("s") * _NC + lax.axis_index("c")
    pltpu.sync_copy(idx_hbm.at[wid], idx_v)
    base = wid * _B_PER_W

    for b in range(_NBUF):
        pltpu.async_copy(table_hbm.at[idx_v.at[b]], bufs[b], sems[b])

    @pl.loop(0, _N_CHUNKS - _NBUF, step=_NBUF)
    def _(g):
        for b in range(_NBUF):
            c = g + b
            pltpu.make_async_copy(table_hbm.at[idx_v.at[c]], bufs[b], sems[b]).wait()
            pltpu.sync_copy(bufs[b], out_hbm.at[pl.ds(base + c * _CHUNK, _CHUNK)])
            pltpu.async_copy(table_hbm.at[idx_v.at[c + _NBUF]], bufs[b], sems[b])

    for b in range(_NBUF):
        c = _N_CHUNKS - _NBUF + b
        pltpu.make_async_copy(table_hbm.at[idx_v.at[c]], bufs[b], sems[b]).wait()
        pltpu.sync_copy(bufs[b], out_hbm.at[pl.ds(base + c * _CHUNK, _CHUNK)])


def kernel(prompt_mask, prompt):
    idx = prompt_mask.astype(jnp.int32).reshape(_NW, _N_CHUNKS, _CHUNK)
    table = prompt.reshape(_POOL_SIZE, _D)
    out = _gather_rows(idx, table)
    return out.reshape(_BATCH, _TOP_K * _LENGTH, _EMBED_DIM)


# speedup vs baseline: 4.5985x; 1.7196x over previous
import functools

import jax
import jax.numpy as jnp
from jax import lax
from jax.experimental import pallas as pl
from jax.experimental.pallas import tpu as pltpu
from jax.experimental.pallas import tpu_sc as plsc

_POOL_SIZE = 1000
_LENGTH = 4
_EMBED_DIM = 128
_BATCH = 16384
_TOP_K = 5

_D = _LENGTH * _EMBED_DIM
_B_TOTAL = _BATCH * _TOP_K
_NC, _NS = 2, 16
_NW = _NC * _NS
_BATCH_PER_W = _BATCH // _NW
_CB = 16
_CHUNK = _CB * _TOP_K
_N_CHUNKS = _BATCH_PER_W // _CB

_mesh = plsc.VectorSubcoreMesh(
    core_axis_name="c", subcore_axis_name="s",
    num_cores=_NC, num_subcores=_NS,
)


_NBUF = 2


@functools.partial(
    pl.kernel,
    out_type=jax.ShapeDtypeStruct((_BATCH, _TOP_K * _LENGTH, _EMBED_DIM), jnp.float32),
    mesh=_mesh,
    scratch_types=[
        pltpu.VMEM((_N_CHUNKS, _CHUNK), jnp.int32),
        [pltpu.VMEM((_CHUNK, _LENGTH, _EMBED_DIM), jnp.float32) for _ in range(_NBUF)],
        [pltpu.SemaphoreType.DMA for _ in range(_NBUF)],
    ],
)
def _gather_rows(idx_hbm, table_hbm, out_hbm, idx_v, bufs, sems):
    wid = lax.axis_index("s") * _NC + lax.axis_index("c")
    pltpu.sync_copy(idx_hbm.at[wid], idx_v)
    base = wid * _BATCH_PER_W

    def write_out(b, c):
        src = bufs[b].reshape(_CB, _TOP_K * _LENGTH, _EMBED_DIM)
        pltpu.sync_copy(src, out_hbm.at[pl.ds(base + c * _CB, _CB)])

    for b in range(_NBUF):
        pltpu.async_copy(table_hbm.at[idx_v.at[b]], bufs[b], sems[b])

    @pl.loop(0, _N_CHUNKS - _NBUF, step=_NBUF)
    def _(g):
        for b in range(_NBUF):
            c = g + b
            pltpu.make_async_copy(table_hbm.at[idx_v.at[c]], bufs[b], sems[b]).wait()
            write_out(b, c)
            pltpu.async_copy(table_hbm.at[idx_v.at[c + _NBUF]], bufs[b], sems[b])

    for b in range(_NBUF):
        c = _N_CHUNKS - _NBUF + b
        pltpu.make_async_copy(table_hbm.at[idx_v.at[c]], bufs[b], sems[b]).wait()
        write_out(b, c)


def kernel(prompt_mask, prompt):
    idx = prompt_mask.astype(jnp.int32).reshape(_NW, _N_CHUNKS, _CHUNK)
    return _gather_rows(idx, prompt)
